# baseline (speedup 1.0000x reference)
import jax
import jax.numpy as jnp
from jax.experimental import pallas as pl

N = 10000
E = 320000
D = 128
EPS = 1e-5


def _mm_body(x_ref, w_ref, o_ref):
    o_ref[...] = jnp.dot(x_ref[...], w_ref[...],
                         preferred_element_type=jnp.float32,
                         precision=jax.lax.Precision.HIGHEST)


def _mm(x, w):
    return pl.pallas_call(
        _mm_body,
        out_shape=jax.ShapeDtypeStruct((x.shape[0], w.shape[1]), jnp.float32),
    )(x, w)


def kernel(in_feat, edge_index, W1, b1, W2, b2, g1, be1, W3, b3, g2, be2):
    src = edge_index[0]
    dst = edge_index[1]
    ones = jnp.ones((E,), dtype=jnp.float32)
    deg_out = jax.ops.segment_sum(ones, src, num_segments=N)
    deg_in = jax.ops.segment_sum(ones, dst, num_segments=N)
    norm_src = jnp.where(deg_out > 0,
                         1.0 / jnp.sqrt(jnp.maximum(deg_out, 1.0)), 0.0)
    norm_dst = jnp.where(deg_in > 0,
                         1.0 / jnp.sqrt(jnp.maximum(deg_in, 1.0)), 0.0)

    def conv(h, W, b):
        hs = h * norm_src[:, None]
        agg = jax.ops.segment_sum(jnp.take(hs, src, axis=0), dst,
                                  num_segments=N)
        return (agg @ W) * norm_dst[:, None] + b

    def bn(h, g, be):
        mean = jnp.mean(h, axis=0)
        var = jnp.var(h, axis=0)
        return (h - mean) / jnp.sqrt(var + EPS) * g + be

    h = jax.nn.relu(conv(in_feat, W1, b1))
    h = jax.nn.relu(conv(h, W2, b2))
    h = bn(h, g1, be1)
    h = jax.nn.relu(conv(h, W3, b3))
    h = bn(h, g2, be2)
    return h


# speedup vs baseline: 3.1594x; 3.1594x over previous
import dataclasses
import functools

import jax
import jax.numpy as jnp
from jax import lax
from jax.experimental import pallas as pl
from jax.experimental.pallas import tpu as pltpu
from jax.experimental.pallas import tpu_sc as plsc

N = 10000
E = 320000
D = 128
EPS = 1e-5

NC = 2
NS = 16
NW = NC * NS

K = 128
CPT = 80
EPM = NW * CPT * K
RPT = 640

NP = 10240
HR = 79

_MESH = plsc.VectorSubcoreMesh(core_axis_name="c", subcore_axis_name="s")

_NO_LAYOUT_CP = pltpu.CompilerParams()
if "needs_layout_passes" in pltpu.CompilerParams.__dataclass_fields__:
    _NO_LAYOUT_CP = dataclasses.replace(_NO_LAYOUT_CP,
                                        needs_layout_passes=False)


@functools.partial(
    pl.kernel,
    out_type=jax.ShapeDtypeStruct((2 * NP, D), jnp.float32),
    mesh=_MESH,
    scratch_types=[
        pltpu.VMEM((CPT, K), jnp.int32),
        pltpu.VMEM((CPT, K), jnp.int32),
        pltpu.VMEM((K, D), jnp.float32),
        pltpu.VMEM_SHARED((NP, D), jnp.float32),
        pltpu.SemaphoreType.DMA,
    ],
)
def _msg_pass(hs_hbm, srcT_hbm, dstT_hbm, z_hbm, out_hbm,
              sidx, didx, rows, acc, sem):
    c = lax.axis_index("c")
    s = lax.axis_index("s")
    w = c * NS + s

    pltpu.sync_copy(z_hbm, acc.at[pl.ds(s * RPT, RPT)])
    pltpu.sync_copy(srcT_hbm.at[w], sidx)
    pltpu.sync_copy(dstT_hbm.at[w], didx)
    plsc.subcore_barrier()

    @pl.loop(0, CPT)
    def _(j):
        pltpu.async_copy(hs_hbm.at[sidx.at[j]], rows, sem).wait()
        pltpu.sync_copy(rows, acc.at[didx.at[j]], add=True)

    plsc.subcore_barrier()
    pltpu.sync_copy(acc.at[pl.ds(s * RPT, RPT)],
                    out_hbm.at[pl.ds(c * NP + s * RPT, RPT)])


@functools.partial(
    pl.kernel,
    out_type=jax.ShapeDtypeStruct((2 * NW, HR, K), jnp.float32),
    mesh=_MESH,
    scratch_types=[
        pltpu.VMEM((CPT, K), jnp.int32),
        pltpu.VMEM((CPT, K), jnp.int32),
        pltpu.VMEM((HR, K), jnp.float32),
        pltpu.VMEM((HR, K), jnp.float32),
    ],
    compiler_params=_NO_LAYOUT_CP,
)
def _degrees(srcT_hbm, dstT_hbm, out_hbm, sidx, didx, hsrc, hdst):
    c = lax.axis_index("c")
    s = lax.axis_index("s")
    w = c * NS + s

    pltpu.sync_copy(srcT_hbm.at[w], sidx)
    pltpu.sync_copy(dstT_hbm.at[w], didx)

    zv = jnp.zeros((16,), jnp.float32)

    @pl.loop(0, HR)
    def _(r):
        @pl.loop(0, K, step=16)
        def _(i):
            hsrc[r, pl.ds(i, 16)] = zv
            hdst[r, pl.ds(i, 16)] = zv

    ones_v = jnp.ones((16,), jnp.float32)

    @pl.loop(0, CPT)
    def _(j):
        @pl.loop(0, K, step=16)
        def _(i):
            sv = sidx[j, pl.ds(i, 16)]
            plsc.addupdate_scatter(
                hsrc, [lax.shift_right_logical(sv, 7),
                       lax.bitwise_and(sv, 127)], ones_v)
            dv = didx[j, pl.ds(i, 16)]
            plsc.addupdate_scatter(
                hdst, [lax.shift_right_logical(dv, 7),
                       lax.bitwise_and(dv, 127)], ones_v)

    pltpu.sync_copy(hsrc, out_hbm.at[w])
    pltpu.sync_copy(hdst, out_hbm.at[NW + w])


def _scale_pad_body(x_ref, ns_ref, o_ref):
    o_ref[:N, :] = x_ref[...] * ns_ref[...]
    o_ref[N:, :] = jnp.zeros((8, D), jnp.float32)


def _scale_pad(x, ns):
    return pl.pallas_call(
        _scale_pad_body,
        out_shape=jax.ShapeDtypeStruct((N + 8, D), jnp.float32),
    )(x, ns)


def _mid_body(bn, p0_ref, p1_ref, w_ref, b_ref, nd_ref, ns_ref, g_ref,
              be_ref, o_ref):
    agg = p0_ref[...] + p1_ref[...]
    y = jnp.dot(agg, w_ref[...], preferred_element_type=jnp.float32)
    y = jnp.maximum(y * nd_ref[...] + b_ref[...], 0.0)
    if bn:
        m = jnp.mean(y, axis=0, keepdims=True)
        v = jnp.mean((y - m) ** 2, axis=0, keepdims=True)
        y = (y - m) / jnp.sqrt(v + EPS) * g_ref[...] + be_ref[...]
    o_ref[:N, :] = y * ns_ref[...]
    o_ref[N:, :] = jnp.zeros((8, D), jnp.float32)


def _mid(parts, w, b, nd, ns, g, be, bn):
    return pl.pallas_call(
        functools.partial(_mid_body, bn),
        out_shape=jax.ShapeDtypeStruct((N + 8, D), jnp.float32),
    )(parts[:N], parts[NP:NP + N], w, b, nd, ns, g, be)


def _final_body(p0_ref, p1_ref, w_ref, b_ref, nd_ref, g_ref, be_ref, o_ref):
    agg = p0_ref[...] + p1_ref[...]
    y = jnp.dot(agg, w_ref[...], preferred_element_type=jnp.float32)
    y = jnp.maximum(y * nd_ref[...] + b_ref[...], 0.0)
    m = jnp.mean(y, axis=0, keepdims=True)
    v = jnp.mean((y - m) ** 2, axis=0, keepdims=True)
    o_ref[...] = (y - m) / jnp.sqrt(v + EPS) * g_ref[...] + be_ref[...]


def _final(parts, w, b, nd, g, be):
    return pl.pallas_call(
        _final_body,
        out_shape=jax.ShapeDtypeStruct((N, D), jnp.float32),
    )(parts[:N], parts[NP:NP + N], w, b, nd, g, be)


def kernel(in_feat, edge_index, W1, b1, W2, b2, g1, be1, W3, b3, g2, be2):
    src = edge_index[0]
    dst = edge_index[1]

    pad = jnp.full((EPM - E,), N, jnp.int32)
    srcp = jnp.concatenate([src, pad])
    dstp = jnp.concatenate([dst, pad])
    srcT = srcp.reshape(NW, CPT, K)
    dstT = dstp.reshape(NW, CPT, K)
    zrows = jnp.zeros((RPT, D), jnp.float32)

    degs = _degrees(srcT, dstT)
    degs = degs.reshape(2, NW, HR * K)
    deg_out = jnp.sum(degs[0], axis=0)[:N]
    deg_in = jnp.sum(degs[1], axis=0)[:N]
    norm_src = jnp.where(deg_out > 0,
                         1.0 / jnp.sqrt(jnp.maximum(deg_out, 1.0)),
                         0.0).reshape(N, 1)
    norm_dst = jnp.where(deg_in > 0,
                         1.0 / jnp.sqrt(jnp.maximum(deg_in, 1.0)),
                         0.0).reshape(N, 1)

    b1r = b1.reshape(1, D)
    b2r = b2.reshape(1, D)
    b3r = b3.reshape(1, D)
    g1r = g1.reshape(1, D)
    be1r = be1.reshape(1, D)
    g2r = g2.reshape(1, D)
    be2r = be2.reshape(1, D)

    hs1 = _scale_pad(in_feat, norm_src)
    p1 = _msg_pass(hs1, srcT, dstT, zrows)
    hs2 = _mid(p1, W1, b1r, norm_dst, norm_src, g1r, be1r, bn=False)
    p2 = _msg_pass(hs2, srcT, dstT, zrows)
    hs3 = _mid(p2, W2, b2r, norm_dst, norm_src, g1r, be1r, bn=True)
    p3 = _msg_pass(hs3, srcT, dstT, zrows)
    return _final(p3, W3, b3r, norm_dst, g2r, be2r)


# speedup vs baseline: 8.4676x; 2.6801x over previous
import dataclasses
import functools

import jax
import jax.numpy as jnp
from jax import lax
from jax.experimental import pallas as pl
from jax.experimental.pallas import tpu as pltpu
from jax.experimental.pallas import tpu_sc as plsc

N = 10000
E = 320000
D = 128
EPS = 1e-5

NC = 2
NS = 16
NW = NC * NS

K = 128
CPT = 80
EPM = NW * CPT * K
RPT = 640

NP = 10240
HR = 79
NTP = N + 128

_MESH = plsc.VectorSubcoreMesh(core_axis_name="c", subcore_axis_name="s")

_NO_LAYOUT_CP = pltpu.CompilerParams()
if "needs_layout_passes" in pltpu.CompilerParams.__dataclass_fields__:
    _NO_LAYOUT_CP = dataclasses.replace(_NO_LAYOUT_CP,
                                        needs_layout_passes=False)


@functools.partial(
    pl.kernel,
    out_type=jax.ShapeDtypeStruct((2 * NP, D), jnp.float32),
    mesh=_MESH,
    scratch_types=[
        pltpu.VMEM((CPT, K), jnp.int32),
        pltpu.VMEM((CPT, K), jnp.int32),
        pltpu.VMEM((K, D), jnp.float32),
        pltpu.VMEM_SHARED((NP, D), jnp.float32),
        pltpu.SemaphoreType.DMA,
    ],
)
def _msg_pass(hs_hbm, srcT_hbm, dstT_hbm, z_hbm, out_hbm,
              sidx, didx, rows, acc, sem):
    c = lax.axis_index("c")
    s = lax.axis_index("s")
    w = c * NS + s

    pltpu.sync_copy(z_hbm, acc.at[pl.ds(s * RPT, RPT)])
    pltpu.sync_copy(srcT_hbm.at[w], sidx)
    pltpu.sync_copy(dstT_hbm.at[w], didx)
    plsc.subcore_barrier()

    @pl.loop(0, CPT)
    def _(j):
        pltpu.async_copy(hs_hbm.at[sidx.at[j]], rows, sem).wait()
        pltpu.sync_copy(rows, acc.at[didx.at[j]], add=True)

    plsc.subcore_barrier()
    pltpu.sync_copy(acc.at[pl.ds(s * RPT, RPT)],
                    out_hbm.at[pl.ds(c * NP + s * RPT, RPT)])


@functools.partial(
    pl.kernel,
    out_type=jax.ShapeDtypeStruct((2 * NW, HR, K), jnp.float32),
    mesh=_MESH,
    scratch_types=[
        pltpu.VMEM((CPT, K), jnp.int32),
        pltpu.VMEM((CPT, K), jnp.int32),
        pltpu.VMEM((HR, K), jnp.float32),
        pltpu.VMEM((HR, K), jnp.float32),
    ],
    compiler_params=_NO_LAYOUT_CP,
)
def _degrees(srcT_hbm, dstT_hbm, out_hbm, sidx, didx, hsrc, hdst):
    c = lax.axis_index("c")
    s = lax.axis_index("s")
    w = c * NS + s

    pltpu.sync_copy(srcT_hbm.at[w], sidx)
    pltpu.sync_copy(dstT_hbm.at[w], didx)

    zv = jnp.zeros((16,), jnp.float32)

    @pl.loop(0, HR)
    def _(r):
        @pl.loop(0, K, step=16)
        def _(i):
            hsrc[r, pl.ds(i, 16)] = zv
            hdst[r, pl.ds(i, 16)] = zv

    ones_v = jnp.ones((16,), jnp.float32)

    @pl.loop(0, CPT)
    def _(j):
        @pl.loop(0, K, step=16)
        def _(i):
            sv = sidx[j, pl.ds(i, 16)]
            plsc.addupdate_scatter(
                hsrc, [lax.shift_right_logical(sv, 7),
                       lax.bitwise_and(sv, 127)], ones_v)
            dv = didx[j, pl.ds(i, 16)]
            plsc.addupdate_scatter(
                hdst, [lax.shift_right_logical(dv, 7),
                       lax.bitwise_and(dv, 127)], ones_v)

    pltpu.sync_copy(hsrc, out_hbm.at[w])
    pltpu.sync_copy(hdst, out_hbm.at[NW + w])


def _scale_pad_body(x_ref, ns_ref, o_ref):
    o_ref[:N, :] = x_ref[...] * ns_ref[...]
    o_ref[N:, :] = jnp.zeros((NTP - N, D), jnp.float32)


def _scale_pad(x, ns):
    return pl.pallas_call(
        _scale_pad_body,
        out_shape=jax.ShapeDtypeStruct((NTP, D), jnp.float32),
    )(x, ns)


def _mid_body(bn, p0_ref, p1_ref, w_ref, b_ref, nd_ref, ns_ref, g_ref,
              be_ref, o_ref):
    agg = p0_ref[...] + p1_ref[...]
    y = jnp.dot(agg, w_ref[...], preferred_element_type=jnp.float32)
    y = jnp.maximum(y * nd_ref[...] + b_ref[...], 0.0)
    if bn:
        m = jnp.mean(y, axis=0, keepdims=True)
        v = jnp.mean((y - m) ** 2, axis=0, keepdims=True)
        y = (y - m) / jnp.sqrt(v + EPS) * g_ref[...] + be_ref[...]
    o_ref[:N, :] = y * ns_ref[...]
    o_ref[N:, :] = jnp.zeros((NTP - N, D), jnp.float32)


def _mid(parts, w, b, nd, ns, g, be, bn):
    return pl.pallas_call(
        functools.partial(_mid_body, bn),
        out_shape=jax.ShapeDtypeStruct((NTP, D), jnp.float32),
    )(parts[:N], parts[NP:NP + N], w, b, nd, ns, g, be)


def _final_body(p0_ref, p1_ref, w_ref, b_ref, nd_ref, g_ref, be_ref, o_ref):
    agg = p0_ref[...] + p1_ref[...]
    y = jnp.dot(agg, w_ref[...], preferred_element_type=jnp.float32)
    y = jnp.maximum(y * nd_ref[...] + b_ref[...], 0.0)
    m = jnp.mean(y, axis=0, keepdims=True)
    v = jnp.mean((y - m) ** 2, axis=0, keepdims=True)
    o_ref[...] = (y - m) / jnp.sqrt(v + EPS) * g_ref[...] + be_ref[...]


def _final(parts, w, b, nd, g, be):
    return pl.pallas_call(
        _final_body,
        out_shape=jax.ShapeDtypeStruct((N, D), jnp.float32),
    )(parts[:N], parts[NP:NP + N], w, b, nd, g, be)


def kernel(in_feat, edge_index, W1, b1, W2, b2, g1, be1, W3, b3, g2, be2):
    src = edge_index[0]
    dst = edge_index[1]

    npad = EPM - E
    pad_idx = N + (jnp.arange(npad, dtype=jnp.int32) % 112)
    pad_src = pad_idx
    pad_dst = pad_idx
    srcT = jnp.concatenate([src, pad_src]).reshape(NW, CPT, K)
    dstT = jnp.concatenate([dst, pad_dst]).reshape(NW, CPT, K)
    zrows = jnp.zeros((RPT, D), jnp.float32)

    degs = _degrees(srcT, dstT)
    degs = degs.reshape(2, NW, HR * K)
    deg_out = jnp.sum(degs[0], axis=0)[:N]
    deg_in = jnp.sum(degs[1], axis=0)[:N]
    norm_src = jnp.where(deg_out > 0,
                         1.0 / jnp.sqrt(jnp.maximum(deg_out, 1.0)),
                         0.0).reshape(N, 1)
    norm_dst = jnp.where(deg_in > 0,
                         1.0 / jnp.sqrt(jnp.maximum(deg_in, 1.0)),
                         0.0).reshape(N, 1)

    b1r = b1.reshape(1, D)
    b2r = b2.reshape(1, D)
    b3r = b3.reshape(1, D)
    g1r = g1.reshape(1, D)
    be1r = be1.reshape(1, D)
    g2r = g2.reshape(1, D)
    be2r = be2.reshape(1, D)

    hs1 = _scale_pad(in_feat, norm_src)
    p1 = _msg_pass(hs1, srcT, dstT, zrows)
    hs2 = _mid(p1, W1, b1r, norm_dst, norm_src, g1r, be1r, bn=False)
    p2 = _msg_pass(hs2, srcT, dstT, zrows)
    hs3 = _mid(p2, W2, b2r, norm_dst, norm_src, g1r, be1r, bn=True)
    p3 = _msg_pass(hs3, srcT, dstT, zrows)
    return _final(p3, W3, b3r, norm_dst, g2r, be2r)


# speedup vs baseline: 12.2751x; 1.4497x over previous
import dataclasses
import functools

import jax
import jax.numpy as jnp
from jax import lax
from jax.experimental import pallas as pl
from jax.experimental.pallas import tpu as pltpu
from jax.experimental.pallas import tpu_sc as plsc

N = 10000
E = 320000
D = 128
EPS = 1e-5

NC = 2
NS = 16
NW = NC * NS

K = 128
CPT = 80
CH = 40
EPM = NW * CPT * K
RPT = 640

NP = 10240
KD = 128
CPTD = EPM // (NW * KD)
HR = 79
NTP = N + 128

_MESH = plsc.VectorSubcoreMesh(core_axis_name="c", subcore_axis_name="s")

_NO_LAYOUT_CP = pltpu.CompilerParams()
if "needs_layout_passes" in pltpu.CompilerParams.__dataclass_fields__:
    _NO_LAYOUT_CP = dataclasses.replace(_NO_LAYOUT_CP,
                                        needs_layout_passes=False)


@functools.partial(
    pl.kernel,
    out_type=jax.ShapeDtypeStruct((2 * NP, D), jnp.float32),
    mesh=_MESH,
    scratch_types=[
        pltpu.VMEM((CH, K), jnp.int32),
        pltpu.VMEM((CH, K), jnp.int32),
        pltpu.VMEM((K, D), jnp.float32),
        pltpu.VMEM((K, D), jnp.float32),
        pltpu.VMEM_SHARED((NP, D), jnp.float32),
        pltpu.SemaphoreType.DMA,
        pltpu.SemaphoreType.DMA,
    ],
)
def _msg_pass(hs_hbm, srcT_hbm, dstT_hbm, z_hbm, out_hbm,
              sidx, didx, rows0, rows1, acc, sem0, sem1):
    c = lax.axis_index("c")
    s = lax.axis_index("s")
    w = c * NS + s

    pltpu.sync_copy(z_hbm, acc.at[pl.ds(s * RPT, RPT)])
    plsc.subcore_barrier()

    dummy = z_hbm.at[pl.ds(0, K)]

    for h in range(2):
        pltpu.sync_copy(srcT_hbm.at[w, pl.ds(h * CH, CH)], sidx)
        pltpu.sync_copy(dstT_hbm.at[w, pl.ds(h * CH, CH)], didx)
        pltpu.async_copy(hs_hbm.at[sidx.at[0]], rows0, sem0)
        pltpu.async_copy(hs_hbm.at[sidx.at[1]], rows1, sem1)

        @pl.loop(0, CH, step=2)
        def _(j):
            pltpu.make_async_copy(dummy, rows0, sem0).wait()
            pltpu.sync_copy(rows0, acc.at[didx.at[j]], add=True)

            @pl.when(j + 2 < CH)
            def _():
                pltpu.async_copy(hs_hbm.at[sidx.at[j + 2]], rows0, sem0)

            pltpu.make_async_copy(dummy, rows1, sem1).wait()
            pltpu.sync_copy(rows1, acc.at[didx.at[j + 1]], add=True)

            @pl.when(j + 3 < CH)
            def _():
                pltpu.async_copy(hs_hbm.at[sidx.at[j + 3]], rows1, sem1)

    plsc.subcore_barrier()
    pltpu.sync_copy(acc.at[pl.ds(s * RPT, RPT)],
                    out_hbm.at[pl.ds(c * NP + s * RPT, RPT)])


@functools.partial(
    pl.kernel,
    out_type=jax.ShapeDtypeStruct((2 * NW, HR, KD), jnp.float32),
    mesh=_MESH,
    scratch_types=[
        pltpu.VMEM((CPTD, KD), jnp.int32),
        pltpu.VMEM((CPTD, KD), jnp.int32),
        pltpu.VMEM((HR, KD), jnp.float32),
        pltpu.VMEM((HR, KD), jnp.float32),
    ],
    compiler_params=_NO_LAYOUT_CP,
)
def _degrees(srcT_hbm, dstT_hbm, out_hbm, sidx, didx, hsrc, hdst):
    c = lax.axis_index("c")
    s = lax.axis_index("s")
    w = c * NS + s

    pltpu.sync_copy(srcT_hbm.at[w], sidx)
    pltpu.sync_copy(dstT_hbm.at[w], didx)

    zv = jnp.zeros((16,), jnp.float32)

    @pl.loop(0, HR)
    def _(r):
        @pl.loop(0, KD, step=16)
        def _(i):
            hsrc[r, pl.ds(i, 16)] = zv
            hdst[r, pl.ds(i, 16)] = zv

    ones_v = jnp.ones((16,), jnp.float32)

    @pl.loop(0, CPTD)
    def _(j):
        @pl.loop(0, KD, step=16)
        def _(i):
            sv = sidx[j, pl.ds(i, 16)]
            plsc.addupdate_scatter(
                hsrc, [lax.shift_right_logical(sv, 7),
                       lax.bitwise_and(sv, 127)], ones_v)
            dv = didx[j, pl.ds(i, 16)]
            plsc.addupdate_scatter(
                hdst, [lax.shift_right_logical(dv, 7),
                       lax.bitwise_and(dv, 127)], ones_v)

    pltpu.sync_copy(hsrc, out_hbm.at[w])
    pltpu.sync_copy(hdst, out_hbm.at[NW + w])


def _scale_pad_body(x_ref, ns_ref, o_ref):
    o_ref[:N, :] = x_ref[...] * ns_ref[...]
    o_ref[N:, :] = jnp.zeros((NTP - N, D), jnp.float32)


def _scale_pad(x, ns):
    return pl.pallas_call(
        _scale_pad_body,
        out_shape=jax.ShapeDtypeStruct((NTP, D), jnp.float32),
    )(x, ns)


def _mid_body(bn, p0_ref, p1_ref, w_ref, b_ref, nd_ref, ns_ref, g_ref,
              be_ref, o_ref):
    agg = p0_ref[...] + p1_ref[...]
    y = jnp.dot(agg, w_ref[...], preferred_element_type=jnp.float32)
    y = jnp.maximum(y * nd_ref[...] + b_ref[...], 0.0)
    if bn:
        m = jnp.mean(y, axis=0, keepdims=True)
        v = jnp.mean((y - m) ** 2, axis=0, keepdims=True)
        y = (y - m) / jnp.sqrt(v + EPS) * g_ref[...] + be_ref[...]
    o_ref[:N, :] = y * ns_ref[...]
    o_ref[N:, :] = jnp.zeros((NTP - N, D), jnp.float32)


def _mid(parts, w, b, nd, ns, g, be, bn):
    return pl.pallas_call(
        functools.partial(_mid_body, bn),
        out_shape=jax.ShapeDtypeStruct((NTP, D), jnp.float32),
    )(parts[:N], parts[NP:NP + N], w, b, nd, ns, g, be)


def _final_body(p0_ref, p1_ref, w_ref, b_ref, nd_ref, g_ref, be_ref, o_ref):
    agg = p0_ref[...] + p1_ref[...]
    y = jnp.dot(agg, w_ref[...], preferred_element_type=jnp.float32)
    y = jnp.maximum(y * nd_ref[...] + b_ref[...], 0.0)
    m = jnp.mean(y, axis=0, keepdims=True)
    v = jnp.mean((y - m) ** 2, axis=0, keepdims=True)
    o_ref[...] = (y - m) / jnp.sqrt(v + EPS) * g_ref[...] + be_ref[...]


def _final(parts, w, b, nd, g, be):
    return pl.pallas_call(
        _final_body,
        out_shape=jax.ShapeDtypeStruct((N, D), jnp.float32),
    )(parts[:N], parts[NP:NP + N], w, b, nd, g, be)


def kernel(in_feat, edge_index, W1, b1, W2, b2, g1, be1, W3, b3, g2, be2):
    src = edge_index[0]
    dst = edge_index[1]

    npad = EPM - E
    pad_idx = N + (jnp.arange(npad, dtype=jnp.int32) % 112)
    pad_src = pad_idx
    pad_dst = pad_idx
    srcT = jnp.concatenate([src, pad_src]).reshape(NW, CPT, K)
    dstT = jnp.concatenate([dst, pad_dst]).reshape(NW, CPT, K)
    zrows = jnp.zeros((RPT, D), jnp.float32)

    degs = _degrees(srcT.reshape(NW, CPTD, KD), dstT.reshape(NW, CPTD, KD))
    degs = degs.reshape(2, NW, HR * KD)
    deg_out = jnp.sum(degs[0], axis=0)[:N]
    deg_in = jnp.sum(degs[1], axis=0)[:N]
    norm_src = jnp.where(deg_out > 0,
                         1.0 / jnp.sqrt(jnp.maximum(deg_out, 1.0)),
                         0.0).reshape(N, 1)
    norm_dst = jnp.where(deg_in > 0,
                         1.0 / jnp.sqrt(jnp.maximum(deg_in, 1.0)),
                         0.0).reshape(N, 1)

    b1r = b1.reshape(1, D)
    b2r = b2.reshape(1, D)
    b3r = b3.reshape(1, D)
    g1r = g1.reshape(1, D)
    be1r = be1.reshape(1, D)
    g2r = g2.reshape(1, D)
    be2r = be2.reshape(1, D)

    hs1 = _scale_pad(in_feat, norm_src)
    p1 = _msg_pass(hs1, srcT, dstT, zrows)
    hs2 = _mid(p1, W1, b1r, norm_dst, norm_src, g1r, be1r, bn=False)
    p2 = _msg_pass(hs2, srcT, dstT, zrows)
    hs3 = _mid(p2, W2, b2r, norm_dst, norm_src, g1r, be1r, bn=True)
    p3 = _msg_pass(hs3, srcT, dstT, zrows)
    return _final(p3, W3, b3r, norm_dst, g2r, be2r)


# speedup vs baseline: 12.7000x; 1.0346x over previous
import dataclasses
import functools

import jax
import jax.numpy as jnp
from jax import lax
from jax.experimental import pallas as pl
from jax.experimental.pallas import tpu as pltpu
from jax.experimental.pallas import tpu_sc as plsc

N = 10000
E = 320000
D = 128
EPS = 1e-5

NC = 2
NS = 16
NW = NC * NS

K = 128
CPT = 80
CH = 40
EPM = NW * CPT * K
RPT = 640

NP = 10240
KD = 128
CPTD = EPM // (NW * KD)
HR = 79
NTP = N + 128

_MESH = plsc.VectorSubcoreMesh(core_axis_name="c", subcore_axis_name="s")

_NO_LAYOUT_CP = pltpu.CompilerParams()
if "needs_layout_passes" in pltpu.CompilerParams.__dataclass_fields__:
    _NO_LAYOUT_CP = dataclasses.replace(_NO_LAYOUT_CP,
                                        needs_layout_passes=False)


@functools.partial(
    pl.kernel,
    out_type=jax.ShapeDtypeStruct((2 * NP, D), jnp.float32),
    mesh=_MESH,
    scratch_types=[
        pltpu.VMEM((CH, K), jnp.int32),
        pltpu.VMEM((CH, K), jnp.int32),
        pltpu.VMEM((K, D), jnp.float32),
        pltpu.VMEM((K, D), jnp.float32),
        pltpu.VMEM_SHARED((NP, D), jnp.float32),
        pltpu.SemaphoreType.DMA,
        pltpu.SemaphoreType.DMA,
    ],
)
def _msg_pass(hs_hbm, srcT_hbm, dstT_hbm, z_hbm, out_hbm,
              sidx, didx, rows0, rows1, acc, sem0, sem1):
    c = lax.axis_index("c")
    s = lax.axis_index("s")
    w = c * NS + s

    pltpu.sync_copy(z_hbm, acc.at[pl.ds(s * RPT, RPT)])
    plsc.subcore_barrier()

    dummy = z_hbm.at[pl.ds(0, K)]

    for h in range(2):
        pltpu.sync_copy(srcT_hbm.at[w, pl.ds(h * CH, CH)], sidx)
        pltpu.sync_copy(dstT_hbm.at[w, pl.ds(h * CH, CH)], didx)
        pltpu.async_copy(hs_hbm.at[sidx.at[0]], rows0, sem0)
        pltpu.async_copy(hs_hbm.at[sidx.at[1]], rows1, sem1)

        @pl.loop(0, CH, step=2)
        def _(j):
            pltpu.make_async_copy(dummy, rows0, sem0).wait()
            pltpu.sync_copy(rows0, acc.at[didx.at[j]], add=True)

            @pl.when(j + 2 < CH)
            def _():
                pltpu.async_copy(hs_hbm.at[sidx.at[j + 2]], rows0, sem0)

            pltpu.make_async_copy(dummy, rows1, sem1).wait()
            pltpu.sync_copy(rows1, acc.at[didx.at[j + 1]], add=True)

            @pl.when(j + 3 < CH)
            def _():
                pltpu.async_copy(hs_hbm.at[sidx.at[j + 3]], rows1, sem1)

    plsc.subcore_barrier()
    pltpu.sync_copy(acc.at[pl.ds(s * RPT, RPT)],
                    out_hbm.at[pl.ds(c * NP + s * RPT, RPT)])


@functools.partial(
    pl.kernel,
    out_type=jax.ShapeDtypeStruct((2 * NW, HR, KD), jnp.float32),
    mesh=_MESH,
    scratch_types=[
        pltpu.VMEM((CPTD, KD), jnp.int32),
        pltpu.VMEM((CPTD, KD), jnp.int32),
        pltpu.VMEM((HR, KD), jnp.float32),
        pltpu.VMEM((HR, KD), jnp.float32),
    ],
    compiler_params=_NO_LAYOUT_CP,
)
def _degrees(srcT_hbm, dstT_hbm, out_hbm, sidx, didx, hsrc, hdst):
    c = lax.axis_index("c")
    s = lax.axis_index("s")
    w = c * NS + s

    pltpu.sync_copy(srcT_hbm.at[w], sidx)
    pltpu.sync_copy(dstT_hbm.at[w], didx)

    zv = jnp.zeros((16,), jnp.float32)

    @pl.loop(0, HR)
    def _(r):
        @pl.loop(0, KD, step=16)
        def _(i):
            hsrc[r, pl.ds(i, 16)] = zv
            hdst[r, pl.ds(i, 16)] = zv

    ones_v = jnp.ones((16,), jnp.float32)

    @pl.loop(0, CPTD)
    def _(j):
        @pl.loop(0, KD, step=16)
        def _(i):
            sv = sidx[j, pl.ds(i, 16)]
            plsc.addupdate_scatter(
                hsrc, [lax.shift_right_logical(sv, 7),
                       lax.bitwise_and(sv, 127)], ones_v)
            dv = didx[j, pl.ds(i, 16)]
            plsc.addupdate_scatter(
                hdst, [lax.shift_right_logical(dv, 7),
                       lax.bitwise_and(dv, 127)], ones_v)

    pltpu.sync_copy(hsrc, out_hbm.at[w])
    pltpu.sync_copy(hdst, out_hbm.at[NW + w])


def _scale_pad_body(x_ref, ns_ref, o_ref):
    o_ref[:N, :] = x_ref[...] * ns_ref[...]
    o_ref[N:, :] = jnp.zeros((NTP - N, D), jnp.float32)


def _scale_pad(x, ns):
    return pl.pallas_call(
        _scale_pad_body,
        out_shape=jax.ShapeDtypeStruct((NTP, D), jnp.float32),
    )(x, ns)


def _mid_body(bn, p_ref, w_ref, b_ref, nd_ref, ns_ref, g_ref,
              be_ref, o_ref):
    agg = p_ref[:N, :] + p_ref[NP:NP + N, :]
    y = jnp.dot(agg, w_ref[...], preferred_element_type=jnp.float32)
    y = jnp.maximum(y * nd_ref[...] + b_ref[...], 0.0)
    if bn:
        m = jnp.mean(y, axis=0, keepdims=True)
        v = jnp.mean((y - m) ** 2, axis=0, keepdims=True)
        y = (y - m) / jnp.sqrt(v + EPS) * g_ref[...] + be_ref[...]
    o_ref[:N, :] = y * ns_ref[...]
    o_ref[N:, :] = jnp.zeros((NTP - N, D), jnp.float32)


def _mid(parts, w, b, nd, ns, g, be, bn):
    return pl.pallas_call(
        functools.partial(_mid_body, bn),
        out_shape=jax.ShapeDtypeStruct((NTP, D), jnp.float32),
    )(parts, w, b, nd, ns, g, be)


def _final_body(p_ref, w_ref, b_ref, nd_ref, g_ref, be_ref, o_ref):
    agg = p_ref[:N, :] + p_ref[NP:NP + N, :]
    y = jnp.dot(agg, w_ref[...], preferred_element_type=jnp.float32)
    y = jnp.maximum(y * nd_ref[...] + b_ref[...], 0.0)
    m = jnp.mean(y, axis=0, keepdims=True)
    v = jnp.mean((y - m) ** 2, axis=0, keepdims=True)
    o_ref[...] = (y - m) / jnp.sqrt(v + EPS) * g_ref[...] + be_ref[...]


def _final(parts, w, b, nd, g, be):
    return pl.pallas_call(
        _final_body,
        out_shape=jax.ShapeDtypeStruct((N, D), jnp.float32),
    )(parts, w, b, nd, g, be)


def kernel(in_feat, edge_index, W1, b1, W2, b2, g1, be1, W3, b3, g2, be2):
    src = edge_index[0]
    dst = edge_index[1]

    npad = EPM - E
    pad_idx = N + (jnp.arange(npad, dtype=jnp.int32) % 112)
    pad_src = pad_idx
    pad_dst = pad_idx
    srcT = jnp.concatenate([src, pad_src]).reshape(NW, CPT, K)
    dstT = jnp.concatenate([dst, pad_dst]).reshape(NW, CPT, K)
    zrows = jnp.zeros((RPT, D), jnp.float32)

    degs = _degrees(srcT, dstT)
    deg_out = jnp.sum(degs[:NW], axis=0).reshape(HR * KD)[:N]
    deg_in = jnp.sum(degs[NW:], axis=0).reshape(HR * KD)[:N]
    norm_src = jnp.where(deg_out > 0,
                         1.0 / jnp.sqrt(jnp.maximum(deg_out, 1.0)),
                         0.0).reshape(N, 1)
    norm_dst = jnp.where(deg_in > 0,
                         1.0 / jnp.sqrt(jnp.maximum(deg_in, 1.0)),
                         0.0).reshape(N, 1)

    b1r = b1.reshape(1, D)
    b2r = b2.reshape(1, D)
    b3r = b3.reshape(1, D)
    g1r = g1.reshape(1, D)
    be1r = be1.reshape(1, D)
    g2r = g2.reshape(1, D)
    be2r = be2.reshape(1, D)

    hs1 = _scale_pad(in_feat, norm_src)
    p1 = _msg_pass(hs1, srcT, dstT, zrows)
    hs2 = _mid(p1, W1, b1r, norm_dst, norm_src, g1r, be1r, bn=False)
    p2 = _msg_pass(hs2, srcT, dstT, zrows)
    hs3 = _mid(p2, W2, b2r, norm_dst, norm_src, g1r, be1r, bn=True)
    p3 = _msg_pass(hs3, srcT, dstT, zrows)
    return _final(p3, W3, b3r, norm_dst, g2r, be2r)


# speedup vs baseline: 12.8090x; 1.0086x over previous
import dataclasses
import functools

import jax
import jax.numpy as jnp
from jax import lax
from jax.experimental import pallas as pl
from jax.experimental.pallas import tpu as pltpu
from jax.experimental.pallas import tpu_sc as plsc

N = 10000
E = 320000
D = 128
EPS = 1e-5

NC = 2
NS = 16
NW = NC * NS

K = 128
CPT = 80
CH = 40
EPM = NW * CPT * K
RPT = 640

NP = 10240
KD = 128
CPTD = EPM // (NW * KD)
HR = 79
NTP = N + 128

_MESH = plsc.VectorSubcoreMesh(core_axis_name="c", subcore_axis_name="s")

_NO_LAYOUT_CP = pltpu.CompilerParams()
if "needs_layout_passes" in pltpu.CompilerParams.__dataclass_fields__:
    _NO_LAYOUT_CP = dataclasses.replace(_NO_LAYOUT_CP,
                                        needs_layout_passes=False)


@functools.partial(
    pl.kernel,
    out_type=jax.ShapeDtypeStruct((2 * NP, D), jnp.float32),
    mesh=_MESH,
    scratch_types=[
        pltpu.VMEM((CH, K), jnp.int32),
        pltpu.VMEM((CH, K), jnp.int32),
        pltpu.VMEM((K, D), jnp.float32),
        pltpu.VMEM((K, D), jnp.float32),
        pltpu.VMEM_SHARED((NP, D), jnp.float32),
        pltpu.SemaphoreType.DMA,
        pltpu.SemaphoreType.DMA,
    ],
)
def _msg_pass(hs_hbm, srcT_hbm, dstT_hbm, z_hbm, out_hbm,
              sidx, didx, rows0, rows1, acc, sem0, sem1):
    c = lax.axis_index("c")
    s = lax.axis_index("s")
    w = c * NS + s

    zc = pltpu.async_copy(z_hbm, acc.at[pl.ds(s * RPT, RPT)], sem0)
    sc = pltpu.async_copy(srcT_hbm.at[w, pl.ds(0, CH)], sidx, sem1)
    dc = pltpu.async_copy(dstT_hbm.at[w, pl.ds(0, CH)], didx, sem1)
    zc.wait()
    sc.wait()
    dc.wait()
    plsc.subcore_barrier()

    dummy = z_hbm.at[pl.ds(0, K)]

    for h in range(2):
        if h:
            pltpu.sync_copy(srcT_hbm.at[w, pl.ds(h * CH, CH)], sidx)
            pltpu.sync_copy(dstT_hbm.at[w, pl.ds(h * CH, CH)], didx)
        pltpu.async_copy(hs_hbm.at[sidx.at[0]], rows0, sem0)
        pltpu.async_copy(hs_hbm.at[sidx.at[1]], rows1, sem1)

        @pl.loop(0, CH, step=2)
        def _(j):
            pltpu.make_async_copy(dummy, rows0, sem0).wait()
            pltpu.sync_copy(rows0, acc.at[didx.at[j]], add=True)

            @pl.when(j + 2 < CH)
            def _():
                pltpu.async_copy(hs_hbm.at[sidx.at[j + 2]], rows0, sem0)

            pltpu.make_async_copy(dummy, rows1, sem1).wait()
            pltpu.sync_copy(rows1, acc.at[didx.at[j + 1]], add=True)

            @pl.when(j + 3 < CH)
            def _():
                pltpu.async_copy(hs_hbm.at[sidx.at[j + 3]], rows1, sem1)

    plsc.subcore_barrier()
    pltpu.sync_copy(acc.at[pl.ds(s * RPT, RPT)],
                    out_hbm.at[pl.ds(c * NP + s * RPT, RPT)])


@functools.partial(
    pl.kernel,
    out_type=jax.ShapeDtypeStruct((2 * NW, HR, KD), jnp.float32),
    mesh=_MESH,
    scratch_types=[
        pltpu.VMEM((CPTD, KD), jnp.int32),
        pltpu.VMEM((CPTD, KD), jnp.int32),
        pltpu.VMEM((HR, KD), jnp.float32),
        pltpu.VMEM((HR, KD), jnp.float32),
    ],
    compiler_params=_NO_LAYOUT_CP,
)
def _degrees(srcT_hbm, dstT_hbm, out_hbm, sidx, didx, hsrc, hdst):
    c = lax.axis_index("c")
    s = lax.axis_index("s")
    w = c * NS + s

    pltpu.sync_copy(srcT_hbm.at[w], sidx)
    pltpu.sync_copy(dstT_hbm.at[w], didx)

    zv = jnp.zeros((16,), jnp.float32)

    @pl.loop(0, HR)
    def _(r):
        @pl.loop(0, KD, step=16)
        def _(i):
            hsrc[r, pl.ds(i, 16)] = zv
            hdst[r, pl.ds(i, 16)] = zv

    ones_v = jnp.ones((16,), jnp.float32)

    @pl.loop(0, CPTD)
    def _(j):
        @pl.loop(0, KD, step=16)
        def _(i):
            sv = sidx[j, pl.ds(i, 16)]
            plsc.addupdate_scatter(
                hsrc, [lax.shift_right_logical(sv, 7),
                       lax.bitwise_and(sv, 127)], ones_v)
            dv = didx[j, pl.ds(i, 16)]
            plsc.addupdate_scatter(
                hdst, [lax.shift_right_logical(dv, 7),
                       lax.bitwise_and(dv, 127)], ones_v)

    pltpu.sync_copy(hsrc, out_hbm.at[w])
    pltpu.sync_copy(hdst, out_hbm.at[NW + w])


def _scale_pad_body(x_ref, ns_ref, o_ref):
    o_ref[:N, :] = x_ref[...] * ns_ref[...]
    o_ref[N:, :] = jnp.zeros((NTP - N, D), jnp.float32)


def _scale_pad(x, ns):
    return pl.pallas_call(
        _scale_pad_body,
        out_shape=jax.ShapeDtypeStruct((NTP, D), jnp.float32),
    )(x, ns)


def _mid_body(bn, p_ref, w_ref, b_ref, nd_ref, ns_ref, g_ref,
              be_ref, o_ref):
    agg = p_ref[:N, :] + p_ref[NP:NP + N, :]
    y = jnp.dot(agg, w_ref[...], preferred_element_type=jnp.float32)
    y = jnp.maximum(y * nd_ref[...] + b_ref[...], 0.0)
    if bn:
        m = jnp.mean(y, axis=0, keepdims=True)
        v = jnp.mean((y - m) ** 2, axis=0, keepdims=True)
        y = (y - m) / jnp.sqrt(v + EPS) * g_ref[...] + be_ref[...]
    o_ref[:N, :] = y * ns_ref[...]
    o_ref[N:, :] = jnp.zeros((NTP - N, D), jnp.float32)


def _mid(parts, w, b, nd, ns, g, be, bn):
    return pl.pallas_call(
        functools.partial(_mid_body, bn),
        out_shape=jax.ShapeDtypeStruct((NTP, D), jnp.float32),
    )(parts, w, b, nd, ns, g, be)


def _final_body(p_ref, w_ref, b_ref, nd_ref, g_ref, be_ref, o_ref):
    agg = p_ref[:N, :] + p_ref[NP:NP + N, :]
    y = jnp.dot(agg, w_ref[...], preferred_element_type=jnp.float32)
    y = jnp.maximum(y * nd_ref[...] + b_ref[...], 0.0)
    m = jnp.mean(y, axis=0, keepdims=True)
    v = jnp.mean((y - m) ** 2, axis=0, keepdims=True)
    o_ref[...] = (y - m) / jnp.sqrt(v + EPS) * g_ref[...] + be_ref[...]


def _final(parts, w, b, nd, g, be):
    return pl.pallas_call(
        _final_body,
        out_shape=jax.ShapeDtypeStruct((N, D), jnp.float32),
    )(parts, w, b, nd, g, be)


def kernel(in_feat, edge_index, W1, b1, W2, b2, g1, be1, W3, b3, g2, be2):
    src = edge_index[0]
    dst = edge_index[1]

    npad = EPM - E
    pad_idx = N + (jnp.arange(npad, dtype=jnp.int32) % 112)
    pad_src = pad_idx
    pad_dst = pad_idx
    srcT = jnp.concatenate([src, pad_src]).reshape(NW, CPT, K)
    dstT = jnp.concatenate([dst, pad_dst]).reshape(NW, CPT, K)
    zrows = jnp.zeros((RPT, D), jnp.float32)

    degs = _degrees(srcT, dstT)
    deg_out = jnp.sum(degs[:NW], axis=0).reshape(HR * KD)[:N]
    deg_in = jnp.sum(degs[NW:], axis=0).reshape(HR * KD)[:N]
    norm_src = jnp.where(deg_out > 0,
                         1.0 / jnp.sqrt(jnp.maximum(deg_out, 1.0)),
                         0.0).reshape(N, 1)
    norm_dst = jnp.where(deg_in > 0,
                         1.0 / jnp.sqrt(jnp.maximum(deg_in, 1.0)),
                         0.0).reshape(N, 1)

    b1r = b1.reshape(1, D)
    b2r = b2.reshape(1, D)
    b3r = b3.reshape(1, D)
    g1r = g1.reshape(1, D)
    be1r = be1.reshape(1, D)
    g2r = g2.reshape(1, D)
    be2r = be2.reshape(1, D)

    hs1 = _scale_pad(in_feat, norm_src)
    p1 = _msg_pass(hs1, srcT, dstT, zrows)
    hs2 = _mid(p1, W1, b1r, norm_dst, norm_src, g1r, be1r, bn=False)
    p2 = _msg_pass(hs2, srcT, dstT, zrows)
    hs3 = _mid(p2, W2, b2r, norm_dst, norm_src, g1r, be1r, bn=True)
    p3 = _msg_pass(hs3, srcT, dstT, zrows)
    return _final(p3, W3, b3r, norm_dst, g2r, be2r)
